# baseline (speedup 1.0000x reference)
import functools

import jax
import jax.numpy as jnp
from jax import lax
from jax.experimental import pallas as pl
from jax.experimental.pallas import tpu as pltpu
from jax.experimental.pallas import tpu_sc as plsc

N = 10000
NP = 10112
E = 320000
F = 128
NC = 2
NS = 16
NW = NC * NS
CHUNK = 128
CPW = 80
GCH = 16
GROUPS = CPW // GCH
EP = NW * CPW * CHUNK
ROWS_PER_SUB = NP // NS

def _degrees_body(src_hbm, dst_hbm, ones_hbm, zeros_hbm, deg_hbm,
                  src_t, dst_t, ones_t, deg_s):
    cid = lax.axis_index("c")
    sid = lax.axis_index("s")
    wid = sid * NC + cid
    r0 = sid * ROWS_PER_SUB
    pltpu.sync_copy(zeros_hbm.at[pl.ds(r0, ROWS_PER_SUB)],
                    deg_s.at[pl.ds(r0, ROWS_PER_SUB)])
    pltpu.sync_copy(ones_hbm, ones_t)
    plsc.subcore_barrier()

    @pl.loop(0, GROUPS)
    def _(g):
        base = wid * CPW + g * GCH
        pltpu.sync_copy(src_hbm.at[pl.ds(base, GCH)], src_t)
        pltpu.sync_copy(dst_hbm.at[pl.ds(base, GCH)], dst_t)

        @pl.loop(0, GCH)
        def _(j):
            pltpu.sync_copy(ones_t.at[pl.ds(0, CHUNK)], deg_s.at[src_t.at[j]],
                            add=True)
            pltpu.sync_copy(ones_t.at[pl.ds(CHUNK, CHUNK)],
                            deg_s.at[dst_t.at[j]], add=True)

    plsc.subcore_barrier()
    pltpu.sync_copy(deg_s.at[pl.ds(r0, ROWS_PER_SUB)],
                    deg_hbm.at[cid].at[pl.ds(r0, ROWS_PER_SUB)])


def _edge_agg_body(h_hbm, src_hbm, dst_hbm, zeros_hbm, out_hbm,
                   src_t, dst_t, rows0, rows1, acc_s, sem0, sem1, ssem0, ssem1):
    cid = lax.axis_index("c")
    sid = lax.axis_index("s")
    wid = sid * NC + cid
    r0 = sid * ROWS_PER_SUB
    pltpu.sync_copy(zeros_hbm.at[pl.ds(r0, ROWS_PER_SUB)],
                    acc_s.at[pl.ds(r0, ROWS_PER_SUB)])
    plsc.subcore_barrier()

    @pl.loop(0, GROUPS)
    def _(g):
        base = wid * CPW + g * GCH
        pltpu.sync_copy(src_hbm.at[pl.ds(base, GCH)], src_t)
        pltpu.sync_copy(dst_hbm.at[pl.ds(base, GCH)], dst_t)
        pltpu.make_async_copy(h_hbm.at[src_t.at[0]], rows0, sem0).start()
        pltpu.make_async_copy(h_hbm.at[src_t.at[1]], rows1, sem1).start()

        @pl.loop(0, GCH, step=2)
        def _(j):
            pltpu.make_async_copy(h_hbm.at[src_t.at[j]], rows0, sem0).wait()
            s0 = pltpu.async_copy(rows0, acc_s.at[dst_t.at[j]], ssem0,
                                  add=True)
            pltpu.make_async_copy(h_hbm.at[src_t.at[j + 1]], rows1, sem1).wait()
            s1 = pltpu.async_copy(rows1, acc_s.at[dst_t.at[j + 1]], ssem1,
                                  add=True)
            s0.wait()

            @pl.when(j + 2 < GCH)
            def _():
                pltpu.make_async_copy(h_hbm.at[src_t.at[j + 2]], rows0,
                                      sem0).start()

            s1.wait()

            @pl.when(j + 3 < GCH)
            def _():
                pltpu.make_async_copy(h_hbm.at[src_t.at[j + 3]], rows1,
                                      sem1).start()

    plsc.subcore_barrier()
    pltpu.sync_copy(acc_s.at[pl.ds(r0, ROWS_PER_SUB)],
                    out_hbm.at[cid].at[pl.ds(r0, ROWS_PER_SUB)])


def _norms_body(deg_ref, no_ref, ni_ref):
    dego = deg_ref[0, :, 0:1] + deg_ref[1, :, 0:1]
    degi = deg_ref[0, :, 64:65] + deg_ref[1, :, 64:65]
    zeros = jnp.zeros((NP, F), jnp.float32)
    no_ref[...] = zeros + jnp.where(
        dego > 0, lax.rsqrt(jnp.maximum(dego, 1.0)), 0.0)
    ni_ref[...] = zeros + jnp.where(
        degi > 0, lax.rsqrt(jnp.maximum(degi, 1.0)), 0.0)


def _stage_in_body(x_ref, w_ref, no_ref, h_ref):
    h = jnp.dot(x_ref[...], w_ref[...], preferred_element_type=jnp.float32)
    h_ref[...] = h * no_ref[...]


def _stage_mid_body(p_ref, ni_ref, no_ref, b_ref, w_ref, h_ref):
    agg = p_ref[0] + p_ref[1]
    y = jnp.maximum(agg * ni_ref[...] + b_ref[...], 0.0)
    h = jnp.dot(y, w_ref[...], preferred_element_type=jnp.float32)
    h_ref[...] = h * no_ref[...]


def _stage_out_body(p_ref, ni_ref, b_ref, o_ref):
    agg = p_ref[0] + p_ref[1]
    o_ref[...] = agg * ni_ref[...] + b_ref[...]


_norms = pl.pallas_call(
    _norms_body,
    out_shape=(jax.ShapeDtypeStruct((NP, F), jnp.float32),
               jax.ShapeDtypeStruct((NP, F), jnp.float32)),
)
_stage_in = pl.pallas_call(
    _stage_in_body,
    out_shape=jax.ShapeDtypeStruct((NP, F), jnp.float32),
)
_stage_mid = pl.pallas_call(
    _stage_mid_body,
    out_shape=jax.ShapeDtypeStruct((NP, F), jnp.float32),
)
_stage_out = pl.pallas_call(
    _stage_out_body,
    out_shape=jax.ShapeDtypeStruct((NP, F), jnp.float32),
)


@functools.cache
def _sc_kernels():
    mesh = plsc.VectorSubcoreMesh(core_axis_name="c", subcore_axis_name="s",
                                  num_cores=NC, num_subcores=NS)
    degrees = pl.kernel(
        _degrees_body,
        out_type=jax.ShapeDtypeStruct((NC, NP, F), jnp.float32),
        mesh=mesh,
        scratch_types=[
            pltpu.VMEM((GCH, CHUNK), jnp.int32),
            pltpu.VMEM((GCH, CHUNK), jnp.int32),
            pltpu.VMEM((2 * CHUNK, F), jnp.float32),
            pltpu.VMEM_SHARED((NP, F), jnp.float32),
        ],
    )
    edge_agg = pl.kernel(
        _edge_agg_body,
        out_type=jax.ShapeDtypeStruct((NC, NP, F), jnp.float32),
        mesh=mesh,
        scratch_types=[
            pltpu.VMEM((GCH, CHUNK), jnp.int32),
            pltpu.VMEM((GCH, CHUNK), jnp.int32),
            pltpu.VMEM((CHUNK, F), jnp.float32),
            pltpu.VMEM((CHUNK, F), jnp.float32),
            pltpu.VMEM_SHARED((NP, F), jnp.float32),
            pltpu.SemaphoreType.DMA,
            pltpu.SemaphoreType.DMA,
            pltpu.SemaphoreType.DMA,
            pltpu.SemaphoreType.DMA,
        ],
    )
    return degrees, edge_agg


def kernel(inputs, edge_index, W1, b1, W2, b2, W3, b3):
    _degrees, _edge_agg = _sc_kernels()
    src = edge_index[0].astype(jnp.int32)
    dst = edge_index[1].astype(jnp.int32)
    pad_e = EP - E
    src_p = jnp.concatenate(
        [src, jnp.full((pad_e,), N, jnp.int32)]).reshape(NW * CPW, CHUNK)
    dst_p = jnp.concatenate(
        [dst, jnp.full((pad_e,), N, jnp.int32)]).reshape(NW * CPW, CHUNK)
    x_p = jnp.pad(inputs, ((0, NP - N), (0, 0)))
    zeros_f = jnp.zeros((NP, F), jnp.float32)
    ones_f = jnp.concatenate([
        jnp.tile(jnp.eye(1, F, 0, dtype=jnp.float32), (CHUNK, 1)),
        jnp.tile(jnp.eye(1, F, 64, dtype=jnp.float32), (CHUNK, 1)),
    ])

    deg = _degrees(src_p, dst_p, ones_f, zeros_f)
    norm_o, norm_i = _norms(deg)

    h1 = _stage_in(x_p, W1, norm_o)
    p1 = _edge_agg(h1, src_p, dst_p, zeros_f)
    h2 = _stage_mid(p1, norm_i, norm_o, b1.reshape(1, F), W2)
    p2 = _edge_agg(h2, src_p, dst_p, zeros_f)
    h3 = _stage_mid(p2, norm_i, norm_o, b2.reshape(1, F), W3)
    p3 = _edge_agg(h3, src_p, dst_p, zeros_f)
    out = _stage_out(p3, norm_i, b3.reshape(1, F))
    return out[:N]


# speedup vs baseline: 1.0902x; 1.0902x over previous
import functools

import jax
import jax.numpy as jnp
from jax import lax
from jax.experimental import pallas as pl
from jax.experimental.pallas import tpu as pltpu
from jax.experimental.pallas import tpu_sc as plsc

N = 10000
NP = 10112
E = 320000
F = 128
NC = 2
NS = 16
NW = NC * NS
CHUNK = 128
CPW = 80
GCH = 8
GROUPS = CPW // GCH
CPW0 = 120
CPW1 = 40
NCH0 = NS * CPW0
EP = NW * CPW * CHUNK
ROWS_PER_SUB = NP // NS

def _degrees_body(src_hbm, dst_hbm, ones_hbm, zeros_hbm, deg_hbm,
                  src_t, dst_t, ones_t, deg_s):
    cid = lax.axis_index("c")
    sid = lax.axis_index("s")
    wid = sid * NC + cid
    r0 = sid * ROWS_PER_SUB
    pltpu.sync_copy(zeros_hbm.at[pl.ds(r0, ROWS_PER_SUB)],
                    deg_s.at[pl.ds(r0, ROWS_PER_SUB)])
    pltpu.sync_copy(ones_hbm, ones_t)
    plsc.subcore_barrier()

    @pl.loop(0, GROUPS)
    def _(g):
        base = wid * CPW + g * GCH
        pltpu.sync_copy(src_hbm.at[pl.ds(base, GCH)], src_t)
        pltpu.sync_copy(dst_hbm.at[pl.ds(base, GCH)], dst_t)

        @pl.loop(0, GCH)
        def _(j):
            pltpu.sync_copy(ones_t.at[pl.ds(0, CHUNK)], deg_s.at[src_t.at[j]],
                            add=True)
            pltpu.sync_copy(ones_t.at[pl.ds(CHUNK, CHUNK)],
                            deg_s.at[dst_t.at[j]], add=True)

    plsc.subcore_barrier()
    pltpu.sync_copy(deg_s.at[pl.ds(r0, ROWS_PER_SUB)],
                    deg_hbm.at[cid].at[pl.ds(r0, ROWS_PER_SUB)])


def _edge_agg_body(h_hbm, src_hbm, dst_hbm, zeros_hbm, out_hbm,
                   src_t, dst_t, rows0, rows1, acc_s, sem0, sem1):
    cid = lax.axis_index("c")
    sid = lax.axis_index("s")
    wid = sid * NC + cid
    r0 = sid * ROWS_PER_SUB
    pltpu.sync_copy(zeros_hbm.at[pl.ds(r0, ROWS_PER_SUB)],
                    acc_s.at[pl.ds(r0, ROWS_PER_SUB)])
    plsc.subcore_barrier()

    def run(base_chunk, n_groups):
        @pl.loop(0, n_groups)
        def _(g):
            base = base_chunk + g * GCH
            pltpu.sync_copy(src_hbm.at[pl.ds(base, GCH)], src_t)
            pltpu.sync_copy(dst_hbm.at[pl.ds(base, GCH)], dst_t)
            pltpu.make_async_copy(h_hbm.at[src_t.at[0]], rows0, sem0).start()

            @pl.loop(0, GCH, step=2)
            def _(j):
                pltpu.make_async_copy(h_hbm.at[src_t.at[j + 1]], rows1,
                                      sem1).start()
                pltpu.make_async_copy(h_hbm.at[src_t.at[j]], rows0,
                                      sem0).wait()
                pltpu.sync_copy(rows0, acc_s.at[dst_t.at[j]], add=True)

                @pl.when(j + 2 < GCH)
                def _():
                    pltpu.make_async_copy(h_hbm.at[src_t.at[j + 2]], rows0,
                                          sem0).start()

                pltpu.make_async_copy(h_hbm.at[src_t.at[j + 1]], rows1,
                                      sem1).wait()
                pltpu.sync_copy(rows1, acc_s.at[dst_t.at[j + 1]], add=True)

    @pl.when(cid == 0)
    def _():
        run(sid * CPW0, CPW0 // GCH)

    @pl.when(cid == 1)
    def _():
        run(NCH0 + sid * CPW1, CPW1 // GCH)

    plsc.subcore_barrier()
    pltpu.sync_copy(acc_s.at[pl.ds(r0, ROWS_PER_SUB)],
                    out_hbm.at[cid].at[pl.ds(r0, ROWS_PER_SUB)])


def _norms_body(deg_ref, no_ref, ni_ref):
    dego = deg_ref[0, :, 0:1] + deg_ref[1, :, 0:1]
    degi = deg_ref[0, :, 64:65] + deg_ref[1, :, 64:65]
    zeros = jnp.zeros((NP, F), jnp.float32)
    no_ref[...] = zeros + jnp.where(
        dego > 0, lax.rsqrt(jnp.maximum(dego, 1.0)), 0.0)
    ni_ref[...] = zeros + jnp.where(
        degi > 0, lax.rsqrt(jnp.maximum(degi, 1.0)), 0.0)


def _stage_in_body(x_ref, w_ref, no_ref, h_ref):
    h = jnp.dot(x_ref[...], w_ref[...], preferred_element_type=jnp.float32)
    h_ref[...] = h * no_ref[...]


def _stage_mid_body(p_ref, ni_ref, no_ref, b_ref, w_ref, h_ref):
    agg = p_ref[0] + p_ref[1]
    y = jnp.maximum(agg * ni_ref[...] + b_ref[...], 0.0)
    h = jnp.dot(y, w_ref[...], preferred_element_type=jnp.float32)
    h_ref[...] = h * no_ref[...]


def _stage_out_body(p_ref, ni_ref, b_ref, o_ref):
    agg = p_ref[0] + p_ref[1]
    o_ref[...] = agg * ni_ref[...] + b_ref[...]


_norms = pl.pallas_call(
    _norms_body,
    out_shape=(jax.ShapeDtypeStruct((NP, F), jnp.float32),
               jax.ShapeDtypeStruct((NP, F), jnp.float32)),
)
_stage_in = pl.pallas_call(
    _stage_in_body,
    out_shape=jax.ShapeDtypeStruct((NP, F), jnp.float32),
)
_stage_mid = pl.pallas_call(
    _stage_mid_body,
    out_shape=jax.ShapeDtypeStruct((NP, F), jnp.float32),
)
_stage_out = pl.pallas_call(
    _stage_out_body,
    out_shape=jax.ShapeDtypeStruct((NP, F), jnp.float32),
)


@functools.cache
def _sc_kernels():
    mesh = plsc.VectorSubcoreMesh(core_axis_name="c", subcore_axis_name="s",
                                  num_cores=NC, num_subcores=NS)
    degrees = pl.kernel(
        _degrees_body,
        out_type=jax.ShapeDtypeStruct((NC, NP, F), jnp.float32),
        mesh=mesh,
        scratch_types=[
            pltpu.VMEM((GCH, CHUNK), jnp.int32),
            pltpu.VMEM((GCH, CHUNK), jnp.int32),
            pltpu.VMEM((2 * CHUNK, F), jnp.float32),
            pltpu.VMEM_SHARED((NP, F), jnp.float32),
        ],
    )
    edge_agg = pl.kernel(
        _edge_agg_body,
        out_type=jax.ShapeDtypeStruct((NC, NP, F), jnp.float32),
        mesh=mesh,
        scratch_types=[
            pltpu.VMEM((GCH, CHUNK), jnp.int32),
            pltpu.VMEM((GCH, CHUNK), jnp.int32),
            pltpu.VMEM((CHUNK, F), jnp.float32),
            pltpu.VMEM((CHUNK, F), jnp.float32),
            pltpu.VMEM_SHARED((NP, F), jnp.float32),
            pltpu.SemaphoreType.DMA,
            pltpu.SemaphoreType.DMA,
        ],
    )
    return degrees, edge_agg


def kernel(inputs, edge_index, W1, b1, W2, b2, W3, b3):
    _degrees, _edge_agg = _sc_kernels()
    src = edge_index[0].astype(jnp.int32)
    dst = edge_index[1].astype(jnp.int32)
    pad_e = EP - E
    src_p = jnp.concatenate(
        [src, jnp.full((pad_e,), N, jnp.int32)]).reshape(NW * CPW, CHUNK)
    dst_p = jnp.concatenate(
        [dst, jnp.full((pad_e,), N, jnp.int32)]).reshape(NW * CPW, CHUNK)
    x_p = jnp.pad(inputs, ((0, NP - N), (0, 0)))
    zeros_f = jnp.zeros((NP, F), jnp.float32)
    ones_f = jnp.concatenate([
        jnp.tile(jnp.eye(1, F, 0, dtype=jnp.float32), (CHUNK, 1)),
        jnp.tile(jnp.eye(1, F, 64, dtype=jnp.float32), (CHUNK, 1)),
    ])

    deg = _degrees(src_p, dst_p, ones_f, zeros_f)
    norm_o, norm_i = _norms(deg)

    h1 = _stage_in(x_p, W1, norm_o)
    p1 = _edge_agg(h1, src_p, dst_p, zeros_f)
    h2 = _stage_mid(p1, norm_i, norm_o, b1.reshape(1, F), W2)
    p2 = _edge_agg(h2, src_p, dst_p, zeros_f)
    h3 = _stage_mid(p2, norm_i, norm_o, b2.reshape(1, F), W3)
    p3 = _edge_agg(h3, src_p, dst_p, zeros_f)
    out = _stage_out(p3, norm_i, b3.reshape(1, F))
    return out[:N]


# speedup vs baseline: 1.1518x; 1.0564x over previous
import functools

import jax
import jax.numpy as jnp
from jax import lax
from jax.experimental import pallas as pl
from jax.experimental.pallas import tpu as pltpu
from jax.experimental.pallas import tpu_sc as plsc

N = 10000
NP = 10112
E = 320000
F = 128
NC = 2
NS = 16
NW = NC * NS
CHUNK = 128
CPW = 80
GCH = 16
GROUPS = CPW // GCH
CPW0 = 144
CPW1 = 16
NCH0 = NS * CPW0
EP = NW * CPW * CHUNK
ROWS_PER_SUB = NP // NS

def _degrees_body(src_hbm, dst_hbm, ones_hbm, zeros_hbm, deg_hbm,
                  src_t, dst_t, ones_t, deg_s):
    cid = lax.axis_index("c")
    sid = lax.axis_index("s")
    wid = sid * NC + cid
    r0 = sid * ROWS_PER_SUB
    pltpu.sync_copy(zeros_hbm.at[pl.ds(r0, ROWS_PER_SUB)],
                    deg_s.at[pl.ds(r0, ROWS_PER_SUB)])
    pltpu.sync_copy(ones_hbm, ones_t)
    plsc.subcore_barrier()

    @pl.loop(0, GROUPS)
    def _(g):
        base = wid * CPW + g * GCH
        pltpu.sync_copy(src_hbm.at[pl.ds(base, GCH)], src_t)
        pltpu.sync_copy(dst_hbm.at[pl.ds(base, GCH)], dst_t)

        @pl.loop(0, GCH)
        def _(j):
            pltpu.sync_copy(ones_t.at[pl.ds(0, CHUNK)], deg_s.at[src_t.at[j]],
                            add=True)
            pltpu.sync_copy(ones_t.at[pl.ds(CHUNK, CHUNK)],
                            deg_s.at[dst_t.at[j]], add=True)

    plsc.subcore_barrier()
    pltpu.sync_copy(deg_s.at[pl.ds(r0, ROWS_PER_SUB)],
                    deg_hbm.at[cid].at[pl.ds(r0, ROWS_PER_SUB)])


def _edge_agg_body(h_hbm, src_hbm, dst_hbm, zeros_hbm, out_hbm,
                   src_t, dst_t, rows0, rows1, acc_s, sem0, sem1):
    cid = lax.axis_index("c")
    sid = lax.axis_index("s")
    wid = sid * NC + cid
    r0 = sid * ROWS_PER_SUB
    pltpu.sync_copy(zeros_hbm.at[pl.ds(r0, ROWS_PER_SUB)],
                    acc_s.at[pl.ds(r0, ROWS_PER_SUB)])
    plsc.subcore_barrier()

    def run(base_chunk, n_groups):
        @pl.loop(0, n_groups)
        def _(g):
            base = base_chunk + g * GCH
            pltpu.sync_copy(src_hbm.at[pl.ds(base, GCH)], src_t)
            pltpu.sync_copy(dst_hbm.at[pl.ds(base, GCH)], dst_t)
            pltpu.make_async_copy(h_hbm.at[src_t.at[0]], rows0, sem0).start()

            @pl.loop(0, GCH, step=2)
            def _(j):
                pltpu.make_async_copy(h_hbm.at[src_t.at[j + 1]], rows1,
                                      sem1).start()
                pltpu.make_async_copy(h_hbm.at[src_t.at[j]], rows0,
                                      sem0).wait()
                pltpu.sync_copy(rows0, acc_s.at[dst_t.at[j]], add=True)

                @pl.when(j + 2 < GCH)
                def _():
                    pltpu.make_async_copy(h_hbm.at[src_t.at[j + 2]], rows0,
                                          sem0).start()

                pltpu.make_async_copy(h_hbm.at[src_t.at[j + 1]], rows1,
                                      sem1).wait()
                pltpu.sync_copy(rows1, acc_s.at[dst_t.at[j + 1]], add=True)

    @pl.when(cid == 0)
    def _():
        run(sid * CPW0, CPW0 // GCH)

    @pl.when(cid == 1)
    def _():
        run(NCH0 + sid * CPW1, CPW1 // GCH)

    plsc.subcore_barrier()
    pltpu.sync_copy(acc_s.at[pl.ds(r0, ROWS_PER_SUB)],
                    out_hbm.at[cid].at[pl.ds(r0, ROWS_PER_SUB)])


def _norms_body(deg_ref, no_ref, ni_ref):
    dego = deg_ref[0, :, 0:1] + deg_ref[1, :, 0:1]
    degi = deg_ref[0, :, 64:65] + deg_ref[1, :, 64:65]
    zeros = jnp.zeros((NP, F), jnp.float32)
    no_ref[...] = zeros + jnp.where(
        dego > 0, lax.rsqrt(jnp.maximum(dego, 1.0)), 0.0)
    ni_ref[...] = zeros + jnp.where(
        degi > 0, lax.rsqrt(jnp.maximum(degi, 1.0)), 0.0)


def _stage_in_body(x_ref, w_ref, no_ref, h_ref):
    h = jnp.dot(x_ref[...], w_ref[...], preferred_element_type=jnp.float32)
    h_ref[...] = h * no_ref[...]


def _stage_mid_body(p_ref, ni_ref, no_ref, b_ref, w_ref, h_ref):
    agg = p_ref[0] + p_ref[1]
    y = jnp.maximum(agg * ni_ref[...] + b_ref[...], 0.0)
    h = jnp.dot(y, w_ref[...], preferred_element_type=jnp.float32)
    h_ref[...] = h * no_ref[...]


def _stage_out_body(p_ref, ni_ref, b_ref, o_ref):
    agg = p_ref[0] + p_ref[1]
    o_ref[...] = agg * ni_ref[...] + b_ref[...]


_norms = pl.pallas_call(
    _norms_body,
    out_shape=(jax.ShapeDtypeStruct((NP, F), jnp.float32),
               jax.ShapeDtypeStruct((NP, F), jnp.float32)),
)
_stage_in = pl.pallas_call(
    _stage_in_body,
    out_shape=jax.ShapeDtypeStruct((NP, F), jnp.float32),
)
_stage_mid = pl.pallas_call(
    _stage_mid_body,
    out_shape=jax.ShapeDtypeStruct((NP, F), jnp.float32),
)
_stage_out = pl.pallas_call(
    _stage_out_body,
    out_shape=jax.ShapeDtypeStruct((NP, F), jnp.float32),
)


@functools.cache
def _sc_kernels():
    mesh = plsc.VectorSubcoreMesh(core_axis_name="c", subcore_axis_name="s",
                                  num_cores=NC, num_subcores=NS)
    degrees = pl.kernel(
        _degrees_body,
        out_type=jax.ShapeDtypeStruct((NC, NP, F), jnp.float32),
        mesh=mesh,
        scratch_types=[
            pltpu.VMEM((GCH, CHUNK), jnp.int32),
            pltpu.VMEM((GCH, CHUNK), jnp.int32),
            pltpu.VMEM((2 * CHUNK, F), jnp.float32),
            pltpu.VMEM_SHARED((NP, F), jnp.float32),
        ],
    )
    edge_agg = pl.kernel(
        _edge_agg_body,
        out_type=jax.ShapeDtypeStruct((NC, NP, F), jnp.float32),
        mesh=mesh,
        scratch_types=[
            pltpu.VMEM((GCH, CHUNK), jnp.int32),
            pltpu.VMEM((GCH, CHUNK), jnp.int32),
            pltpu.VMEM((CHUNK, F), jnp.float32),
            pltpu.VMEM((CHUNK, F), jnp.float32),
            pltpu.VMEM_SHARED((NP, F), jnp.float32),
            pltpu.SemaphoreType.DMA,
            pltpu.SemaphoreType.DMA,
        ],
    )
    return degrees, edge_agg


def kernel(inputs, edge_index, W1, b1, W2, b2, W3, b3):
    _degrees, _edge_agg = _sc_kernels()
    src = edge_index[0].astype(jnp.int32)
    dst = edge_index[1].astype(jnp.int32)
    pad_e = EP - E
    src_p = jnp.concatenate(
        [src, jnp.full((pad_e,), N, jnp.int32)]).reshape(NW * CPW, CHUNK)
    dst_p = jnp.concatenate(
        [dst, jnp.full((pad_e,), N, jnp.int32)]).reshape(NW * CPW, CHUNK)
    x_p = jnp.pad(inputs, ((0, NP - N), (0, 0)))
    zeros_f = jnp.zeros((NP, F), jnp.float32)
    ones_f = jnp.concatenate([
        jnp.tile(jnp.eye(1, F, 0, dtype=jnp.float32), (CHUNK, 1)),
        jnp.tile(jnp.eye(1, F, 64, dtype=jnp.float32), (CHUNK, 1)),
    ])

    deg = _degrees(src_p, dst_p, ones_f, zeros_f)
    norm_o, norm_i = _norms(deg)

    h1 = _stage_in(x_p, W1, norm_o)
    p1 = _edge_agg(h1, src_p, dst_p, zeros_f)
    h2 = _stage_mid(p1, norm_i, norm_o, b1.reshape(1, F), W2)
    p2 = _edge_agg(h2, src_p, dst_p, zeros_f)
    h3 = _stage_mid(p2, norm_i, norm_o, b2.reshape(1, F), W3)
    p3 = _edge_agg(h3, src_p, dst_p, zeros_f)
    out = _stage_out(p3, norm_i, b3.reshape(1, F))
    return out[:N]
